# Initial kernel scaffold; baseline (speedup 1.0000x reference)
import jax
import jax.numpy as jnp
from jax.experimental import pallas as pl


def kernel(x, edge_index, edge_attr, ptr, W0, b0, W1, b1, W2, b2, W3, b3, SW0, Sb0, SW1, Sb1, SW2, Sb2, MW0, Mb0, MW1, Mb1, MW2, Mb2):
    raise NotImplementedError("write your pallas kernel here")


# speedup vs baseline: 6.1474x; 6.1474x over previous
import functools

import jax
import jax.numpy as jnp
from jax import lax
from jax.experimental import pallas as pl
from jax.experimental.pallas import tpu as pltpu
from jax.experimental.pallas import tpu_sc as plsc

N = 10000
E = 320000
D = 128
B = 16
GN = 625

NC = 2
NS = 16
D2 = D // NC
PIECES = D2 // 16
CHUNK = 128
CPT = 160
GRP = 10
EP = NS * CPT * CHUNK
NPS = 640
NPAD = NS * NPS


def _sc_agg_body(t_hbm, src_hbm, dst_hbm, w_hbm, out_hbm,
                 src_v, dst_v, rows0, rows1, wb0, wb1,
                 s_sh, gsem0, gsem1, wsem0, wsem1):
    cid = lax.axis_index("c")
    sid = lax.axis_index("s")

    pltpu.sync_copy(src_hbm.at[sid], src_v)
    pltpu.sync_copy(dst_hbm.at[sid], dst_v)

    offv = jnp.full((16,), cid * N, jnp.int32)

    def _off(i, carry):
        for j in range(8):
            sl = pl.ds(j * 16, 16)
            src_v[i, sl] = src_v[i, sl] + offv
        return carry

    lax.fori_loop(0, CPT, _off, 0)

    zero16 = jnp.zeros((16,), jnp.float32)

    def _zb(i, carry):
        rows0[i // PIECES, pl.ds((i % PIECES) * 16, 16)] = zero16
        return carry

    lax.fori_loop(0, CHUNK * PIECES, _zb, 0)
    base = sid * NPS
    for z in range(NPS // CHUNK):
        pltpu.sync_copy(rows0, s_sh.at[pl.ds(base + z * CHUNK, CHUNK)])
    plsc.subcore_barrier()

    pltpu.async_copy(t_hbm.at[src_v.at[0]], rows0, gsem0)
    pltpu.async_copy(w_hbm.at[sid, 0], wb0, wsem0)

    bufs = ((rows0, gsem0, wb0, wsem0), (rows1, gsem1, wb1, wsem1))

    def _group(gi, carry):
        for u in range(GRP):
            c = gi * GRP + u
            rows, gsem, wb, wsem = bufs[u % 2]
            o_rows, o_sem, o_wb, o_wsem = bufs[(u + 1) % 2]

            @pl.when(c + 1 < CPT)
            def _():
                pltpu.async_copy(t_hbm.at[src_v.at[c + 1]], o_rows, o_sem)
                pltpu.async_copy(w_hbm.at[sid, c + 1], o_wb, o_wsem)

            pltpu.make_async_copy(t_hbm.at[src_v.at[c]], rows, gsem).wait()
            pltpu.make_async_copy(w_hbm.at[sid, c], wb, wsem).wait()

            def _scale(e, carry2):
                we = wb[e]
                for j in range(PIECES):
                    sl = pl.ds(j * 16, 16)
                    rows[e, sl] = rows[e, sl] * we
                return carry2

            lax.fori_loop(0, CHUNK, _scale, 0)

            pltpu.sync_copy(rows, s_sh.at[dst_v.at[c]], add=True)
        return carry

    lax.fori_loop(0, CPT // GRP, _group, 0)

    plsc.subcore_barrier()
    pltpu.sync_copy(s_sh.at[pl.ds(base, NPS)], out_hbm.at[cid, pl.ds(base, NPS)])


@functools.cache
def _get_sc_agg():
  return pl.kernel(
    _sc_agg_body,
    out_type=jax.ShapeDtypeStruct((NC, NPAD, D2), jnp.float32),
    mesh=plsc.VectorSubcoreMesh(core_axis_name="c", subcore_axis_name="s",
                                num_cores=NC, num_subcores=NS),
    compiler_params=pltpu.CompilerParams(use_tc_tiling_on_sc=False),
    scratch_types=[
        pltpu.VMEM((CPT, CHUNK), jnp.int32),
        pltpu.VMEM((CPT, CHUNK), jnp.int32),
        pltpu.VMEM((CHUNK, D2), jnp.float32),
        pltpu.VMEM((CHUNK, D2), jnp.float32),
        pltpu.VMEM((CHUNK, 16), jnp.float32),
        pltpu.VMEM((CHUNK, 16), jnp.float32),
        pltpu.VMEM_SHARED((NPAD, D2), jnp.float32),
        pltpu.SemaphoreType.DMA,
        pltpu.SemaphoreType.DMA,
        pltpu.SemaphoreType.DMA,
        pltpu.SemaphoreType.DMA,
    ],
  )


def _sc_agg(t, src_p, dst_p, w_p):
    return _get_sc_agg()(t, src_p, dst_p, w_p)


def _dotT(a, w):
    return lax.dot_general(a.astype(jnp.bfloat16), w.astype(jnp.bfloat16),
                           (((1,), (1,)), ((), ())),
                           preferred_element_type=jnp.float32)


R = 2000
NB = N // R


def _split2(tn):
    return jnp.stack([tn[:, :D2], tn[:, D2:]])


def _tc_q_body(x_ref, w0_ref, degp_ref, dinv_ref, t_ref):
    dt = degp_ref[...] + 1.0
    dinv = jnp.where(dt > 0, lax.rsqrt(jnp.maximum(dt, 1e-12)), 0.0)
    dinv_ref[...] = dinv
    t_ref[...] = _split2(dinv * _dotT(x_ref[...], w0_ref[...]))


def _merge_h(s0_ref, s1_ref, t_ref, dinv_ref, b_ref):
    t2 = t_ref[...]
    u = jnp.concatenate([s0_ref[...] + t2[0], s1_ref[...] + t2[1]], axis=1)
    return jnp.maximum(dinv_ref[...] * u + b_ref[...], 0.0)


def _tc_m_body(s0_ref, s1_ref, t_ref, dinv_ref, b_ref, w_ref, out_ref):
    h = _merge_h(s0_ref, s1_ref, t_ref, dinv_ref, b_ref)
    out_ref[...] = _split2(dinv_ref[...] * _dotT(h, w_ref[...]))


def _tc_m3_body(s0_ref, s1_ref, t_ref, dinv_ref, b_ref, hn_ref):
    h = _merge_h(s0_ref, s1_ref, t_ref, dinv_ref, b_ref)
    ss = jnp.sum(h * h, axis=1, keepdims=True)
    nrm = jnp.maximum(jnp.sqrt(ss), 1e-12)
    hn_ref[...] = h / nrm


def _row_specs():
    sb = pl.BlockSpec((R, D2), lambda i: (i, 0))
    tb = pl.BlockSpec((2, R, D2), lambda i: (0, i, 0))
    db = pl.BlockSpec((R, 1), lambda i: (i, 0))
    bb = pl.BlockSpec((1, D), lambda i: (0, 0))
    return sb, tb, db, bb


def _tc_f_body(g_ref, sw0_ref, sb0_ref, sw1_ref, sb1_ref, sw2_ref, sb2_ref,
               hh_ref):
    g = g_ref[0]
    g = jnp.maximum(_dotT(g, sw0_ref[...]) + sb0_ref[...], 0.0)
    g = jnp.maximum(_dotT(g, sw1_ref[...]) + sb1_ref[...], 0.0)
    g = jnp.maximum(_dotT(g, sw2_ref[...]) + sb2_ref[...], 0.0)
    mask = lax.broadcasted_iota(jnp.int32, (NPS, 1), 0) < GN
    g = jnp.where(mask, g, 0.0)
    gb = g.astype(jnp.bfloat16)
    hh_ref[0] = lax.dot_general(gb, gb, (((0,), (0,)), ((), ())),
                                preferred_element_type=jnp.float32)


def _tc_f2_body(hhf_ref, mw0_ref, mb0_ref, mw1_ref, mb1_ref, mw2_ref,
                mb2_ref, o_ref):
    o1 = jnp.maximum(_dotT(hhf_ref[...], mw0_ref[...]) + mb0_ref[...], 0.0)
    o2 = jnp.maximum(_dotT(o1, mw1_ref[...]) + mb1_ref[...], 0.0)
    o_ref[...] = jnp.maximum(_dotT(o2, mw2_ref[...]) + mb2_ref[...], 0.0)


def kernel(x, edge_index, edge_attr, ptr, W0, b0, W1, b1, W2, b2, W3, b3,
           SW0, Sb0, SW1, Sb1, SW2, Sb2, MW0, Mb0, MW1, Mb1, MW2, Mb2):
    src = edge_index[0].astype(jnp.int32)
    dst = edge_index[1].astype(jnp.int32)
    w = edge_attr[:, 0].astype(jnp.float32)

    pad = EP - E
    src_p = jnp.concatenate([src, jnp.zeros((pad,), jnp.int32)]) \
        .reshape(NS, CPT, CHUNK)
    dst_p = jnp.concatenate([dst, jnp.zeros((pad,), jnp.int32)]) \
        .reshape(NS, CPT, CHUNK)
    w_p = jnp.broadcast_to(
        jnp.concatenate([w, jnp.zeros((pad,), jnp.float32)])
        .reshape(NS, CPT, CHUNK)[..., None], (NS, CPT, CHUNK, 16))

    ones_t = jnp.ones((2 * N, D2), jnp.float32)
    degs = _sc_agg(ones_t, src_p, dst_p, w_p)
    degp = degs[0, :N, 0].reshape(N, 1)

    sb, tb, db, bb = _row_specs()
    wb = pl.BlockSpec((D, D), lambda i: (0, 0))

    dinv, t = pl.pallas_call(
        _tc_q_body,
        grid=(NB,),
        in_specs=[pl.BlockSpec((R, D), lambda i: (i, 0)), wb, db],
        out_specs=[db, tb],
        out_shape=[jax.ShapeDtypeStruct((N, 1), jnp.float32),
                   jax.ShapeDtypeStruct((2, N, D2), jnp.float32)],
    )(x, W0, degp)

    bs = [b0.reshape(1, D), b1.reshape(1, D), b2.reshape(1, D),
          b3.reshape(1, D)]
    Ws = [W1, W2, W3]

    for k in range(3):
        s = _sc_agg(t.reshape(2 * N, D2), src_p, dst_p, w_p)
        t = pl.pallas_call(
            _tc_m_body,
            grid=(NB,),
            in_specs=[sb, sb, tb, db, bb, wb],
            out_specs=tb,
            out_shape=jax.ShapeDtypeStruct((2, N, D2), jnp.float32),
        )(s[0, :N], s[1, :N], t, dinv, bs[k], Ws[k])

    s = _sc_agg(t.reshape(2 * N, D2), src_p, dst_p, w_p)
    hn = pl.pallas_call(
        _tc_m3_body,
        grid=(NB,),
        in_specs=[sb, sb, tb, db, bb],
        out_specs=pl.BlockSpec((R, D), lambda i: (i, 0)),
        out_shape=jax.ShapeDtypeStruct((N, D), jnp.float32),
    )(s[0, :N], s[1, :N], t, dinv, bs[3])

    g_p = jnp.pad(hn.reshape(B, GN, D), ((0, 0), (0, NPS - GN), (0, 0)))

    MW2p = jnp.zeros((D, 32), jnp.float32).at[:2].set(MW2)
    Mb2p = jnp.zeros((D,), jnp.float32).at[:2].set(Mb2)

    full = lambda shp: pl.BlockSpec(shp, lambda b: tuple(0 for _ in shp))
    hh3 = pl.pallas_call(
        _tc_f_body,
        grid=(B,),
        in_specs=[
            pl.BlockSpec((1, NPS, D), lambda b: (b, 0, 0)),
            full((32, D)), full((1, 32)),
            full((32, 32)), full((1, 32)),
            full((32, 32)), full((1, 32)),
        ],
        out_specs=pl.BlockSpec((1, 32, 32), lambda b: (b, 0, 0)),
        out_shape=jax.ShapeDtypeStruct((B, 32, 32), jnp.float32),
    )(g_p, SW0, Sb0.reshape(1, 32), SW1, Sb1.reshape(1, 32),
      SW2, Sb2.reshape(1, 32))

    hhf = hh3.reshape(B, 32 * 32)
    o_pad = pl.pallas_call(
        _tc_f2_body,
        out_shape=jax.ShapeDtypeStruct((B, D), jnp.float32),
    )(hhf, MW0, Mb0.reshape(1, 32), MW1, Mb1.reshape(1, 32),
      MW2p, Mb2p.reshape(1, D))

    return hhf, o_pad[:, :2]


# speedup vs baseline: 7.6857x; 1.2502x over previous
import functools

import jax
import jax.numpy as jnp
from jax import lax
from jax.experimental import pallas as pl
from jax.experimental.pallas import tpu as pltpu
from jax.experimental.pallas import tpu_sc as plsc

N = 10000
E = 320000
D = 128
B = 16
GN = 625

NC = 2
NS = 16
D2 = D // NC
PIECES = D2 // 16
CHUNK = 128
CPT = 160
GRP = 10
EP = NS * CPT * CHUNK
NPS = 640
NPAD = NS * NPS


def _sc_agg_body(t_hbm, src_hbm, dst_hbm, w_hbm, out_hbm,
                 src_v, dst_v, rows0, rows1, wb0, wb1,
                 s_sh, gsem0, gsem1, wsem0, wsem1, ssem0, ssem1):
    cid = lax.axis_index("c")
    sid = lax.axis_index("s")

    pltpu.sync_copy(src_hbm.at[sid], src_v)
    pltpu.sync_copy(dst_hbm.at[sid], dst_v)

    offv = jnp.full((16,), cid * N, jnp.int32)

    def _off(i, carry):
        for j in range(8):
            sl = pl.ds(j * 16, 16)
            src_v[i, sl] = src_v[i, sl] + offv
        return carry

    lax.fori_loop(0, CPT, _off, 0)

    zero16 = jnp.zeros((16,), jnp.float32)

    def _zb(i, carry):
        rows0[i // PIECES, pl.ds((i % PIECES) * 16, 16)] = zero16
        return carry

    lax.fori_loop(0, CHUNK * PIECES, _zb, 0)
    base = sid * NPS
    for z in range(NPS // CHUNK):
        pltpu.sync_copy(rows0, s_sh.at[pl.ds(base + z * CHUNK, CHUNK)])
    plsc.subcore_barrier()

    pltpu.async_copy(t_hbm.at[src_v.at[0]], rows0, gsem0)
    pltpu.async_copy(w_hbm.at[sid, 0], wb0, wsem0)

    bufs = ((rows0, gsem0, wb0, wsem0, ssem0), (rows1, gsem1, wb1, wsem1,
                                                ssem1))

    def _group(gi, carry):
        for u in range(GRP):
            c = gi * GRP + u
            rows, gsem, wb, wsem, ssem = bufs[u % 2]
            o_rows, o_sem, o_wb, o_wsem, o_ssem = bufs[(u + 1) % 2]

            @pl.when(c + 1 < CPT)
            def _():
                @pl.when(c >= 1)
                def _():
                    pltpu.make_async_copy(
                        o_rows, s_sh.at[dst_v.at[c - 1]], o_ssem).wait()
                pltpu.async_copy(t_hbm.at[src_v.at[c + 1]], o_rows, o_sem)
                pltpu.async_copy(w_hbm.at[sid, c + 1], o_wb, o_wsem)

            pltpu.make_async_copy(t_hbm.at[src_v.at[c]], rows, gsem).wait()
            pltpu.make_async_copy(w_hbm.at[sid, c], wb, wsem).wait()

            def _scale(eb, carry2):
                for k in range(4):
                    e = eb * 4 + k
                    we = wb[e]
                    for j in range(PIECES):
                        sl = pl.ds(j * 16, 16)
                        rows[e, sl] = rows[e, sl] * we
                return carry2

            lax.fori_loop(0, CHUNK // 4, _scale, 0)

            pltpu.async_copy(rows, s_sh.at[dst_v.at[c]], ssem, add=True)
        return carry

    lax.fori_loop(0, CPT // GRP, _group, 0)

    pltpu.make_async_copy(rows0, s_sh.at[dst_v.at[CPT - 2]], ssem0).wait()
    pltpu.make_async_copy(rows1, s_sh.at[dst_v.at[CPT - 1]], ssem1).wait()

    plsc.subcore_barrier()
    pltpu.sync_copy(s_sh.at[pl.ds(base, NPS)], out_hbm.at[cid, pl.ds(base, NPS)])


@functools.cache
def _get_sc_agg():
  return pl.kernel(
    _sc_agg_body,
    out_type=jax.ShapeDtypeStruct((NC, NPAD, D2), jnp.float32),
    mesh=plsc.VectorSubcoreMesh(core_axis_name="c", subcore_axis_name="s",
                                num_cores=NC, num_subcores=NS),
    compiler_params=pltpu.CompilerParams(use_tc_tiling_on_sc=False),
    scratch_types=[
        pltpu.VMEM((CPT, CHUNK), jnp.int32),
        pltpu.VMEM((CPT, CHUNK), jnp.int32),
        pltpu.VMEM((CHUNK, D2), jnp.float32),
        pltpu.VMEM((CHUNK, D2), jnp.float32),
        pltpu.VMEM((CHUNK, 16), jnp.float32),
        pltpu.VMEM((CHUNK, 16), jnp.float32),
        pltpu.VMEM_SHARED((NPAD, D2), jnp.float32),
        pltpu.SemaphoreType.DMA,
        pltpu.SemaphoreType.DMA,
        pltpu.SemaphoreType.DMA,
        pltpu.SemaphoreType.DMA,
        pltpu.SemaphoreType.DMA,
        pltpu.SemaphoreType.DMA,
    ],
  )


def _sc_agg(t, src_p, dst_p, w_p):
    return _get_sc_agg()(t, src_p, dst_p, w_p)


HCPT = CPT // 2


def _sc_deg_body(dst_hbm, w_hbm, out_hbm, dsth, wh, zb, deg_sh, sem):
    cid = lax.axis_index("c")
    sid = lax.axis_index("s")
    off = cid * HCPT
    pltpu.sync_copy(dst_hbm.at[sid, pl.ds(off, HCPT)], dsth)
    pltpu.sync_copy(w_hbm.at[sid, pl.ds(off, HCPT)], wh)

    zero16 = jnp.zeros((16,), jnp.float32)

    def _z(i, carry):
        zb[pl.ds(i * 16, 16)] = zero16
        return carry

    lax.fori_loop(0, NPS // 16, _z, 0)
    pltpu.sync_copy(zb, deg_sh.at[pl.ds(sid * NPS, NPS)])
    plsc.subcore_barrier()

    def _grp(g, carry):
        for u in range(8):
            c = g * 8 + u
            pltpu.async_copy(wh.at[c], deg_sh.at[dsth.at[c]], sem, add=True)
        for u in range(8):
            c = g * 8 + u
            pltpu.make_async_copy(wh.at[c], deg_sh.at[dsth.at[c]],
                                  sem).wait()
        return carry

    lax.fori_loop(0, HCPT // 8, _grp, 0)

    plsc.subcore_barrier()
    pltpu.sync_copy(deg_sh.at[pl.ds(sid * NPS, NPS)],
                    out_hbm.at[cid, pl.ds(sid * NPS, NPS)])


@functools.cache
def _get_sc_deg():
  return pl.kernel(
    _sc_deg_body,
    out_type=jax.ShapeDtypeStruct((NC, NPAD), jnp.float32),
    mesh=plsc.VectorSubcoreMesh(core_axis_name="c", subcore_axis_name="s",
                                num_cores=NC, num_subcores=NS),
    compiler_params=pltpu.CompilerParams(use_tc_tiling_on_sc=False),
    scratch_types=[
        pltpu.VMEM((HCPT, CHUNK), jnp.int32),
        pltpu.VMEM((HCPT, CHUNK), jnp.float32),
        pltpu.VMEM((NPS,), jnp.float32),
        pltpu.VMEM_SHARED((NPAD,), jnp.float32),
        pltpu.SemaphoreType.DMA,
    ],
  )


def _dotT(a, w):
    return lax.dot_general(a.astype(jnp.bfloat16), w.astype(jnp.bfloat16),
                           (((1,), (1,)), ((), ())),
                           preferred_element_type=jnp.float32)


R = 2000
NB = N // R


def _split2(tn):
    return jnp.stack([tn[:, :D2], tn[:, D2:]])


def _tc_q_body(x_ref, w0_ref, degp_ref, dinv_ref, t_ref):
    dp = degp_ref[...]
    dt = dp[:, :1] + dp[:, 1:2] + 1.0
    dinv = jnp.where(dt > 0, lax.rsqrt(jnp.maximum(dt, 1e-12)), 0.0)
    dinv_ref[...] = dinv
    t_ref[...] = _split2(dinv * _dotT(x_ref[...], w0_ref[...]))


def _merge_h(s0_ref, s1_ref, t_ref, dinv_ref, b_ref):
    t2 = t_ref[...]
    u = jnp.concatenate([s0_ref[...] + t2[0], s1_ref[...] + t2[1]], axis=1)
    return jnp.maximum(dinv_ref[...] * u + b_ref[...], 0.0)


def _tc_m_body(s0_ref, s1_ref, t_ref, dinv_ref, b_ref, w_ref, out_ref):
    h = _merge_h(s0_ref, s1_ref, t_ref, dinv_ref, b_ref)
    out_ref[...] = _split2(dinv_ref[...] * _dotT(h, w_ref[...]))


def _tc_m3_body(s0_ref, s1_ref, t_ref, dinv_ref, b_ref, hn_ref):
    h = _merge_h(s0_ref, s1_ref, t_ref, dinv_ref, b_ref)
    ss = jnp.sum(h * h, axis=1, keepdims=True)
    nrm = jnp.maximum(jnp.sqrt(ss), 1e-12)
    hn_ref[...] = h / nrm


def _row_specs():
    sb = pl.BlockSpec((R, D2), lambda i: (i, 0))
    tb = pl.BlockSpec((2, R, D2), lambda i: (0, i, 0))
    db = pl.BlockSpec((R, 1), lambda i: (i, 0))
    bb = pl.BlockSpec((1, D), lambda i: (0, 0))
    return sb, tb, db, bb


def _tc_f_body(g_ref, sw0_ref, sb0_ref, sw1_ref, sb1_ref, sw2_ref, sb2_ref,
               hh_ref):
    g = g_ref[0]
    g = jnp.maximum(_dotT(g, sw0_ref[...]) + sb0_ref[...], 0.0)
    g = jnp.maximum(_dotT(g, sw1_ref[...]) + sb1_ref[...], 0.0)
    g = jnp.maximum(_dotT(g, sw2_ref[...]) + sb2_ref[...], 0.0)
    mask = lax.broadcasted_iota(jnp.int32, (NPS, 1), 0) < GN
    g = jnp.where(mask, g, 0.0)
    gb = g.astype(jnp.bfloat16)
    hh_ref[0] = lax.dot_general(gb, gb, (((0,), (0,)), ((), ())),
                                preferred_element_type=jnp.float32)


def _tc_f2_body(hhf_ref, mw0_ref, mb0_ref, mw1_ref, mb1_ref, mw2_ref,
                mb2_ref, o_ref):
    o1 = jnp.maximum(_dotT(hhf_ref[...], mw0_ref[...]) + mb0_ref[...], 0.0)
    o2 = jnp.maximum(_dotT(o1, mw1_ref[...]) + mb1_ref[...], 0.0)
    o_ref[...] = jnp.maximum(_dotT(o2, mw2_ref[...]) + mb2_ref[...], 0.0)


def kernel(x, edge_index, edge_attr, ptr, W0, b0, W1, b1, W2, b2, W3, b3,
           SW0, Sb0, SW1, Sb1, SW2, Sb2, MW0, Mb0, MW1, Mb1, MW2, Mb2):
    src = edge_index[0].astype(jnp.int32)
    dst = edge_index[1].astype(jnp.int32)
    w = edge_attr[:, 0].astype(jnp.float32)

    pad = EP - E
    src_p = jnp.concatenate([src, jnp.zeros((pad,), jnp.int32)]) \
        .reshape(NS, CPT, CHUNK)
    dst_p = jnp.concatenate([dst, jnp.zeros((pad,), jnp.int32)]) \
        .reshape(NS, CPT, CHUNK)
    w_p = jnp.broadcast_to(
        jnp.concatenate([w, jnp.zeros((pad,), jnp.float32)])
        .reshape(NS, CPT, CHUNK)[..., None], (NS, CPT, CHUNK, 16))

    w_p1 = jnp.concatenate([w, jnp.zeros((pad,), jnp.float32)]) \
        .reshape(NS, CPT, CHUNK)
    degs = _get_sc_deg()(dst_p, w_p1)
    degp = jnp.stack([degs[0, :N], degs[1, :N]], axis=1)

    sb, tb, db, bb = _row_specs()
    wb = pl.BlockSpec((D, D), lambda i: (0, 0))

    dinv, t = pl.pallas_call(
        _tc_q_body,
        grid=(NB,),
        in_specs=[pl.BlockSpec((R, D), lambda i: (i, 0)), wb,
                  pl.BlockSpec((R, 2), lambda i: (i, 0))],
        out_specs=[db, tb],
        out_shape=[jax.ShapeDtypeStruct((N, 1), jnp.float32),
                   jax.ShapeDtypeStruct((2, N, D2), jnp.float32)],
    )(x, W0, degp)

    bs = [b0.reshape(1, D), b1.reshape(1, D), b2.reshape(1, D),
          b3.reshape(1, D)]
    Ws = [W1, W2, W3]

    for k in range(3):
        s = _sc_agg(t.reshape(2 * N, D2), src_p, dst_p, w_p)
        t = pl.pallas_call(
            _tc_m_body,
            grid=(NB,),
            in_specs=[sb, sb, tb, db, bb, wb],
            out_specs=tb,
            out_shape=jax.ShapeDtypeStruct((2, N, D2), jnp.float32),
        )(s[0, :N], s[1, :N], t, dinv, bs[k], Ws[k])

    s = _sc_agg(t.reshape(2 * N, D2), src_p, dst_p, w_p)
    hn = pl.pallas_call(
        _tc_m3_body,
        grid=(NB,),
        in_specs=[sb, sb, tb, db, bb],
        out_specs=pl.BlockSpec((R, D), lambda i: (i, 0)),
        out_shape=jax.ShapeDtypeStruct((N, D), jnp.float32),
    )(s[0, :N], s[1, :N], t, dinv, bs[3])

    g_p = jnp.pad(hn.reshape(B, GN, D), ((0, 0), (0, NPS - GN), (0, 0)))

    MW2p = jnp.zeros((D, 32), jnp.float32).at[:2].set(MW2)
    Mb2p = jnp.zeros((D,), jnp.float32).at[:2].set(Mb2)

    full = lambda shp: pl.BlockSpec(shp, lambda b: tuple(0 for _ in shp))
    hh3 = pl.pallas_call(
        _tc_f_body,
        grid=(B,),
        in_specs=[
            pl.BlockSpec((1, NPS, D), lambda b: (b, 0, 0)),
            full((32, D)), full((1, 32)),
            full((32, 32)), full((1, 32)),
            full((32, 32)), full((1, 32)),
        ],
        out_specs=pl.BlockSpec((1, 32, 32), lambda b: (b, 0, 0)),
        out_shape=jax.ShapeDtypeStruct((B, 32, 32), jnp.float32),
    )(g_p, SW0, Sb0.reshape(1, 32), SW1, Sb1.reshape(1, 32),
      SW2, Sb2.reshape(1, 32))

    hhf = hh3.reshape(B, 32 * 32)
    o_pad = pl.pallas_call(
        _tc_f2_body,
        out_shape=jax.ShapeDtypeStruct((B, D), jnp.float32),
    )(hhf, MW0, Mb0.reshape(1, 32), MW1, Mb1.reshape(1, 32),
      MW2p, Mb2p.reshape(1, D))

    return hhf, o_pad[:, :2]


# speedup vs baseline: 7.6952x; 1.0012x over previous
import functools

import jax
import jax.numpy as jnp
from jax import lax
from jax.experimental import pallas as pl
from jax.experimental.pallas import tpu as pltpu
from jax.experimental.pallas import tpu_sc as plsc

N = 10000
E = 320000
D = 128
B = 16
GN = 625

NC = 2
NS = 16
D2 = D // NC
PIECES = D2 // 16
CHUNK = 128
CPT = 160
GRP = 8
EP = NS * CPT * CHUNK
NPS = 640
NPAD = NS * NPS


def _sc_agg_body(t_hbm, src_hbm, dst_hbm, w_hbm, out_hbm,
                 src_v, dst_v, rows0, rows1, rows2, rows3,
                 wb0, wb1, wb2, wb3, s_sh,
                 gsem0, gsem1, gsem2, gsem3,
                 wsem0, wsem1, wsem2, wsem3,
                 ssem0, ssem1, ssem2, ssem3):
    cid = lax.axis_index("c")
    sid = lax.axis_index("s")

    pltpu.sync_copy(src_hbm.at[sid], src_v)
    pltpu.sync_copy(dst_hbm.at[sid], dst_v)

    offv = jnp.full((16,), cid * N, jnp.int32)

    def _off(i, carry):
        for j in range(8):
            sl = pl.ds(j * 16, 16)
            src_v[i, sl] = src_v[i, sl] + offv
        return carry

    lax.fori_loop(0, CPT, _off, 0)

    zero16 = jnp.zeros((16,), jnp.float32)

    def _zb(i, carry):
        rows0[i // PIECES, pl.ds((i % PIECES) * 16, 16)] = zero16
        return carry

    lax.fori_loop(0, CHUNK * PIECES, _zb, 0)
    base = sid * NPS
    for z in range(NPS // CHUNK):
        pltpu.sync_copy(rows0, s_sh.at[pl.ds(base + z * CHUNK, CHUNK)])
    plsc.subcore_barrier()

    bufs = ((rows0, gsem0, wb0, wsem0, ssem0),
            (rows1, gsem1, wb1, wsem1, ssem1),
            (rows2, gsem2, wb2, wsem2, ssem2),
            (rows3, gsem3, wb3, wsem3, ssem3))
    NBUF = 4

    for p in range(NBUF - 1):
        pltpu.async_copy(t_hbm.at[src_v.at[p]], bufs[p][0], bufs[p][1])
        pltpu.async_copy(w_hbm.at[sid, p], bufs[p][2], bufs[p][3])

    def _group(gi, carry):
        for u in range(GRP):
            c = gi * GRP + u
            rows, gsem, wb, wsem, ssem = bufs[u % NBUF]
            o_rows, o_sem, o_wb, o_wsem, o_ssem = bufs[(u + 3) % NBUF]

            @pl.when(c + 3 < CPT)
            def _():
                @pl.when(c >= 1)
                def _():
                    pltpu.make_async_copy(
                        o_rows, s_sh.at[dst_v.at[c - 1]], o_ssem).wait()
                pltpu.async_copy(t_hbm.at[src_v.at[c + 3]], o_rows, o_sem)
                pltpu.async_copy(w_hbm.at[sid, c + 3], o_wb, o_wsem)

            pltpu.make_async_copy(t_hbm.at[src_v.at[c]], rows, gsem).wait()
            pltpu.make_async_copy(w_hbm.at[sid, c], wb, wsem).wait()

            def _scale(eb, carry2):
                for k in range(4):
                    e = eb * 4 + k
                    we = wb[e]
                    for j in range(PIECES):
                        sl = pl.ds(j * 16, 16)
                        rows[e, sl] = rows[e, sl] * we
                return carry2

            lax.fori_loop(0, CHUNK // 4, _scale, 0)

            pltpu.async_copy(rows, s_sh.at[dst_v.at[c]], ssem, add=True)
        return carry

    lax.fori_loop(0, CPT // GRP, _group, 0)

    for p in range(NBUF):
        cc = CPT - NBUF + p
        pltpu.make_async_copy(bufs[cc % NBUF][0], s_sh.at[dst_v.at[cc]],
                              bufs[cc % NBUF][4]).wait()

    plsc.subcore_barrier()
    pltpu.sync_copy(s_sh.at[pl.ds(base, NPS)], out_hbm.at[cid, pl.ds(base, NPS)])


@functools.cache
def _get_sc_agg():
  return pl.kernel(
    _sc_agg_body,
    out_type=jax.ShapeDtypeStruct((NC, NPAD, D2), jnp.float32),
    mesh=plsc.VectorSubcoreMesh(core_axis_name="c", subcore_axis_name="s",
                                num_cores=NC, num_subcores=NS),
    compiler_params=pltpu.CompilerParams(use_tc_tiling_on_sc=False),
    scratch_types=[
        pltpu.VMEM((CPT, CHUNK), jnp.int32),
        pltpu.VMEM((CPT, CHUNK), jnp.int32),
        pltpu.VMEM((CHUNK, D2), jnp.float32),
        pltpu.VMEM((CHUNK, D2), jnp.float32),
        pltpu.VMEM((CHUNK, D2), jnp.float32),
        pltpu.VMEM((CHUNK, D2), jnp.float32),
        pltpu.VMEM((CHUNK, 16), jnp.float32),
        pltpu.VMEM((CHUNK, 16), jnp.float32),
        pltpu.VMEM((CHUNK, 16), jnp.float32),
        pltpu.VMEM((CHUNK, 16), jnp.float32),
        pltpu.VMEM_SHARED((NPAD, D2), jnp.float32),
    ] + [pltpu.SemaphoreType.DMA] * 12,
  )


def _sc_agg(t, src_p, dst_p, w_p):
    return _get_sc_agg()(t, src_p, dst_p, w_p)


HCPT = CPT // 2


def _sc_deg_body(dst_hbm, w_hbm, out_hbm, dsth, wh, zb, deg_sh, sem):
    cid = lax.axis_index("c")
    sid = lax.axis_index("s")
    off = cid * HCPT
    pltpu.sync_copy(dst_hbm.at[sid, pl.ds(off, HCPT)], dsth)
    pltpu.sync_copy(w_hbm.at[sid, pl.ds(off, HCPT)], wh)

    zero16 = jnp.zeros((16,), jnp.float32)

    def _z(i, carry):
        zb[pl.ds(i * 16, 16)] = zero16
        return carry

    lax.fori_loop(0, NPS // 16, _z, 0)
    pltpu.sync_copy(zb, deg_sh.at[pl.ds(sid * NPS, NPS)])
    plsc.subcore_barrier()

    def _grp(g, carry):
        for u in range(8):
            c = g * 8 + u
            pltpu.async_copy(wh.at[c], deg_sh.at[dsth.at[c]], sem, add=True)
        for u in range(8):
            c = g * 8 + u
            pltpu.make_async_copy(wh.at[c], deg_sh.at[dsth.at[c]],
                                  sem).wait()
        return carry

    lax.fori_loop(0, HCPT // 8, _grp, 0)

    plsc.subcore_barrier()
    pltpu.sync_copy(deg_sh.at[pl.ds(sid * NPS, NPS)],
                    out_hbm.at[cid, pl.ds(sid * NPS, NPS)])


@functools.cache
def _get_sc_deg():
  return pl.kernel(
    _sc_deg_body,
    out_type=jax.ShapeDtypeStruct((NC, NPAD), jnp.float32),
    mesh=plsc.VectorSubcoreMesh(core_axis_name="c", subcore_axis_name="s",
                                num_cores=NC, num_subcores=NS),
    compiler_params=pltpu.CompilerParams(use_tc_tiling_on_sc=False),
    scratch_types=[
        pltpu.VMEM((HCPT, CHUNK), jnp.int32),
        pltpu.VMEM((HCPT, CHUNK), jnp.float32),
        pltpu.VMEM((NPS,), jnp.float32),
        pltpu.VMEM_SHARED((NPAD,), jnp.float32),
        pltpu.SemaphoreType.DMA,
    ],
  )


def _dotT(a, w):
    return lax.dot_general(a.astype(jnp.bfloat16), w.astype(jnp.bfloat16),
                           (((1,), (1,)), ((), ())),
                           preferred_element_type=jnp.float32)


R = 2000
NB = N // R


def _split2(tn):
    return jnp.stack([tn[:, :D2], tn[:, D2:]])


def _tc_q_body(x_ref, w0_ref, degp_ref, dinv_ref, t_ref):
    dp = degp_ref[...]
    dt = dp[:, :1] + dp[:, 1:2] + 1.0
    dinv = jnp.where(dt > 0, lax.rsqrt(jnp.maximum(dt, 1e-12)), 0.0)
    dinv_ref[...] = dinv
    t_ref[...] = _split2(dinv * _dotT(x_ref[...], w0_ref[...]))


def _merge_h(s0_ref, s1_ref, t_ref, dinv_ref, b_ref):
    t2 = t_ref[...]
    u = jnp.concatenate([s0_ref[...] + t2[0], s1_ref[...] + t2[1]], axis=1)
    return jnp.maximum(dinv_ref[...] * u + b_ref[...], 0.0)


def _tc_m_body(s0_ref, s1_ref, t_ref, dinv_ref, b_ref, w_ref, out_ref):
    h = _merge_h(s0_ref, s1_ref, t_ref, dinv_ref, b_ref)
    out_ref[...] = _split2(dinv_ref[...] * _dotT(h, w_ref[...]))


def _tc_m3_body(s0_ref, s1_ref, t_ref, dinv_ref, b_ref, hn_ref):
    h = _merge_h(s0_ref, s1_ref, t_ref, dinv_ref, b_ref)
    ss = jnp.sum(h * h, axis=1, keepdims=True)
    nrm = jnp.maximum(jnp.sqrt(ss), 1e-12)
    hn_ref[...] = h / nrm


def _row_specs():
    sb = pl.BlockSpec((R, D2), lambda i: (i, 0))
    tb = pl.BlockSpec((2, R, D2), lambda i: (0, i, 0))
    db = pl.BlockSpec((R, 1), lambda i: (i, 0))
    bb = pl.BlockSpec((1, D), lambda i: (0, 0))
    return sb, tb, db, bb


def _tc_f_body(g_ref, sw0_ref, sb0_ref, sw1_ref, sb1_ref, sw2_ref, sb2_ref,
               hh_ref):
    g = g_ref[0]
    g = jnp.maximum(_dotT(g, sw0_ref[...]) + sb0_ref[...], 0.0)
    g = jnp.maximum(_dotT(g, sw1_ref[...]) + sb1_ref[...], 0.0)
    g = jnp.maximum(_dotT(g, sw2_ref[...]) + sb2_ref[...], 0.0)
    mask = lax.broadcasted_iota(jnp.int32, (NPS, 1), 0) < GN
    g = jnp.where(mask, g, 0.0)
    gb = g.astype(jnp.bfloat16)
    hh_ref[0] = lax.dot_general(gb, gb, (((0,), (0,)), ((), ())),
                                preferred_element_type=jnp.float32)


def _tc_f2_body(hhf_ref, mw0_ref, mb0_ref, mw1_ref, mb1_ref, mw2_ref,
                mb2_ref, o_ref):
    o1 = jnp.maximum(_dotT(hhf_ref[...], mw0_ref[...]) + mb0_ref[...], 0.0)
    o2 = jnp.maximum(_dotT(o1, mw1_ref[...]) + mb1_ref[...], 0.0)
    o_ref[...] = jnp.maximum(_dotT(o2, mw2_ref[...]) + mb2_ref[...], 0.0)


def kernel(x, edge_index, edge_attr, ptr, W0, b0, W1, b1, W2, b2, W3, b3,
           SW0, Sb0, SW1, Sb1, SW2, Sb2, MW0, Mb0, MW1, Mb1, MW2, Mb2):
    src = edge_index[0].astype(jnp.int32)
    dst = edge_index[1].astype(jnp.int32)
    w = edge_attr[:, 0].astype(jnp.float32)

    pad = EP - E
    src_p = jnp.concatenate([src, jnp.zeros((pad,), jnp.int32)]) \
        .reshape(NS, CPT, CHUNK)
    dst_p = jnp.concatenate([dst, jnp.zeros((pad,), jnp.int32)]) \
        .reshape(NS, CPT, CHUNK)
    w_p = jnp.broadcast_to(
        jnp.concatenate([w, jnp.zeros((pad,), jnp.float32)])
        .reshape(NS, CPT, CHUNK)[..., None], (NS, CPT, CHUNK, 16))

    w_p1 = jnp.concatenate([w, jnp.zeros((pad,), jnp.float32)]) \
        .reshape(NS, CPT, CHUNK)
    degs = _get_sc_deg()(dst_p, w_p1)
    degp = jnp.stack([degs[0, :N], degs[1, :N]], axis=1)

    sb, tb, db, bb = _row_specs()
    wb = pl.BlockSpec((D, D), lambda i: (0, 0))

    dinv, t = pl.pallas_call(
        _tc_q_body,
        grid=(NB,),
        in_specs=[pl.BlockSpec((R, D), lambda i: (i, 0)), wb,
                  pl.BlockSpec((R, 2), lambda i: (i, 0))],
        out_specs=[db, tb],
        out_shape=[jax.ShapeDtypeStruct((N, 1), jnp.float32),
                   jax.ShapeDtypeStruct((2, N, D2), jnp.float32)],
    )(x, W0, degp)

    bs = [b0.reshape(1, D), b1.reshape(1, D), b2.reshape(1, D),
          b3.reshape(1, D)]
    Ws = [W1, W2, W3]

    for k in range(3):
        s = _sc_agg(t.reshape(2 * N, D2), src_p, dst_p, w_p)
        t = pl.pallas_call(
            _tc_m_body,
            grid=(NB,),
            in_specs=[sb, sb, tb, db, bb, wb],
            out_specs=tb,
            out_shape=jax.ShapeDtypeStruct((2, N, D2), jnp.float32),
        )(s[0, :N], s[1, :N], t, dinv, bs[k], Ws[k])

    s = _sc_agg(t.reshape(2 * N, D2), src_p, dst_p, w_p)
    hn = pl.pallas_call(
        _tc_m3_body,
        grid=(NB,),
        in_specs=[sb, sb, tb, db, bb],
        out_specs=pl.BlockSpec((R, D), lambda i: (i, 0)),
        out_shape=jax.ShapeDtypeStruct((N, D), jnp.float32),
    )(s[0, :N], s[1, :N], t, dinv, bs[3])

    g_p = jnp.pad(hn.reshape(B, GN, D), ((0, 0), (0, NPS - GN), (0, 0)))

    MW2p = jnp.zeros((D, 32), jnp.float32).at[:2].set(MW2)
    Mb2p = jnp.zeros((D,), jnp.float32).at[:2].set(Mb2)

    full = lambda shp: pl.BlockSpec(shp, lambda b: tuple(0 for _ in shp))
    hh3 = pl.pallas_call(
        _tc_f_body,
        grid=(B,),
        in_specs=[
            pl.BlockSpec((1, NPS, D), lambda b: (b, 0, 0)),
            full((32, D)), full((1, 32)),
            full((32, 32)), full((1, 32)),
            full((32, 32)), full((1, 32)),
        ],
        out_specs=pl.BlockSpec((1, 32, 32), lambda b: (b, 0, 0)),
        out_shape=jax.ShapeDtypeStruct((B, 32, 32), jnp.float32),
    )(g_p, SW0, Sb0.reshape(1, 32), SW1, Sb1.reshape(1, 32),
      SW2, Sb2.reshape(1, 32))

    hhf = hh3.reshape(B, 32 * 32)
    o_pad = pl.pallas_call(
        _tc_f2_body,
        out_shape=jax.ShapeDtypeStruct((B, D), jnp.float32),
    )(hhf, MW0, Mb0.reshape(1, 32), MW1, Mb1.reshape(1, 32),
      MW2p, Mb2p.reshape(1, D))

    return hhf, o_pad[:, :2]
